# baseline (speedup 1.0000x reference)
import jax
import jax.numpy as jnp
from jax.experimental import pallas as pl

_D = 64
_E = 1024
_BETA = 0.25
_T = 512


def _vq_body(xt_ref, emb_ref, rn_ref, cn_ref, q_ref, idx_ref, loss_ref):
    i = pl.program_id(0)
    xt = xt_ref[...]
    emb = emb_ref[...]
    sim = jnp.dot(xt, emb)
    rn = rn_ref[...]
    cn = cn_ref[...]
    d = (rn + cn) - 2.0 * sim
    dmin = jnp.min(d, axis=1, keepdims=True)
    iota = jax.lax.broadcasted_iota(jnp.int32, d.shape, 1)
    idx = jnp.min(jnp.where(d == dmin, iota, d.shape[1]), axis=1)
    onehot = (iota == idx[:, None]).astype(jnp.float32)
    q = jax.lax.dot_general(onehot, emb, (((1,), (1,)), ((), ())),
                            precision=jax.lax.Precision.HIGHEST)
    q_ref[...] = q
    idx_ref[...] = idx[:, None].astype(jnp.int32)
    part = jnp.sum(dmin, axis=0, keepdims=True)

    @pl.when(i == 0)
    def _():
        loss_ref[...] = jnp.zeros_like(part)

    loss_ref[...] += part


def kernel(x, embeddings):
    input_shape = x.shape
    flat = x.reshape(-1, _D)
    m = flat.shape[0]
    grid = m // _T
    rownorm = jnp.sum(flat ** 2, axis=1, keepdims=True)
    colnorm = jnp.sum(embeddings ** 2, axis=0, keepdims=True)
    q, _idx, loss_sum = pl.pallas_call(
        _vq_body,
        grid=(grid,),
        in_specs=[
            pl.BlockSpec((_T, _D), lambda i: (i, 0)),
            pl.BlockSpec((_D, _E), lambda i: (0, 0)),
            pl.BlockSpec((_T, 1), lambda i: (i, 0)),
            pl.BlockSpec((1, _E), lambda i: (0, 0)),
        ],
        out_specs=[
            pl.BlockSpec((_T, _D), lambda i: (i, 0)),
            pl.BlockSpec((_T, 1), lambda i: (i, 0)),
            pl.BlockSpec((1, 1), lambda i: (0, 0)),
        ],
        out_shape=[
            jax.ShapeDtypeStruct((m, _D), jnp.float32),
            jax.ShapeDtypeStruct((m, 1), jnp.int32),
            jax.ShapeDtypeStruct((1, 1), jnp.float32),
        ],
    )(flat, embeddings, rownorm, colnorm)
    quantized = q.reshape(input_shape)
    loss = loss_sum[0, 0] * ((1.0 + _BETA) / (m * _D))
    return quantized, loss


# speedup vs baseline: 1.6245x; 1.6245x over previous
import jax
import jax.numpy as jnp
from jax.experimental import pallas as pl

_D = 64
_E = 1024
_BETA = 0.25
_T = 512


def _vq_body(xt_ref, emb_ref, rn_ref, cn_ref, q_ref, idx_ref, loss_ref):
    i = pl.program_id(0)
    xt = xt_ref[...]
    emb = emb_ref[...]
    sim = jnp.dot(xt, emb)
    rn = rn_ref[...]
    cn = cn_ref[...]
    d = (rn + cn) - 2.0 * sim
    dmin = jnp.min(d, axis=1, keepdims=True)
    iota = jax.lax.broadcasted_iota(jnp.int32, d.shape, 1)
    idx = jnp.min(jnp.where(d == dmin, iota, d.shape[1]), axis=1)
    onehot = (iota == idx[:, None]).astype(jnp.float32)
    q = jax.lax.dot_general(onehot, emb, (((1,), (1,)), ((), ())))
    q_ref[...] = q
    idx_ref[...] = idx[:, None].astype(jnp.int32)
    part = jnp.sum(dmin, axis=0, keepdims=True)

    @pl.when(i == 0)
    def _():
        loss_ref[...] = jnp.zeros_like(part)

    loss_ref[...] += part


def kernel(x, embeddings):
    input_shape = x.shape
    flat = x.reshape(-1, _D)
    m = flat.shape[0]
    grid = m // _T
    rownorm = jnp.sum(flat ** 2, axis=1, keepdims=True)
    colnorm = jnp.sum(embeddings ** 2, axis=0, keepdims=True)
    q, _idx, loss_sum = pl.pallas_call(
        _vq_body,
        grid=(grid,),
        in_specs=[
            pl.BlockSpec((_T, _D), lambda i: (i, 0)),
            pl.BlockSpec((_D, _E), lambda i: (0, 0)),
            pl.BlockSpec((_T, 1), lambda i: (i, 0)),
            pl.BlockSpec((1, _E), lambda i: (0, 0)),
        ],
        out_specs=[
            pl.BlockSpec((_T, _D), lambda i: (i, 0)),
            pl.BlockSpec((_T, 1), lambda i: (i, 0)),
            pl.BlockSpec((1, 1), lambda i: (0, 0)),
        ],
        out_shape=[
            jax.ShapeDtypeStruct((m, _D), jnp.float32),
            jax.ShapeDtypeStruct((m, 1), jnp.int32),
            jax.ShapeDtypeStruct((1, 1), jnp.float32),
        ],
    )(flat, embeddings, rownorm, colnorm)
    quantized = q.reshape(input_shape)
    loss = loss_sum[0, 0] * ((1.0 + _BETA) / (m * _D))
    return quantized, loss
